# baseline (speedup 1.0000x reference)
import jax
import jax.numpy as jnp
import numpy as np
from jax import lax
from jax.experimental import pallas as pl
from jax.experimental.pallas import tpu as pltpu
from jax.experimental.pallas import tpu_sc as plsc

_D = 256
_N = 8192
_NC = 2
_NS = 16
_NW = _NC * _NS
_L = 16
_ROWS = _N // _NW
_CHUNK = 128
_NCHUNK = _ROWS // _CHUNK
_DCH = _D // _L
_IMAX = np.int32(2**31 - 1)


def _lane15(x, fifteen):
    return x.at[fifteen].get(mode="promise_in_bounds")


def _sc_body(emb_hbm, cent_hbm, sim_hbm, idx_hbm,
             e_v, buf0, buf1, st_f, st_i, sem0, sem1):
    cid = lax.axis_index("c")
    sid = lax.axis_index("s")
    wid = cid * _NS + sid
    base = wid * _ROWS

    pltpu.sync_copy(emb_hbm, e_v)

    fifteen = jnp.broadcast_to(np.int32(_L - 1), (_L,))

    ech = [e_v[pl.ds(j * _L, _L)] for j in range(_DCH)]

    a_b = jnp.full((_L,), -jnp.inf, jnp.float32)
    sq_b = jnp.ones((_L,), jnp.float32)
    bx = jnp.zeros((_L,), jnp.int32)

    bufs = (buf0, buf1)
    sems = (sem0, sem1)
    copies = [None, None]
    copies[0] = pltpu.async_copy(cent_hbm.at[pl.ds(base, _CHUNK)], buf0, sem0)
    for ci in range(_NCHUNK):
        if ci + 1 < _NCHUNK:
            nxt = (ci + 1) % 2
            copies[nxt] = pltpu.async_copy(
                cent_hbm.at[pl.ds(base + (ci + 1) * _CHUNK, _CHUNK)],
                bufs[nxt], sems[nxt])
        copies[ci % 2].wait()
        buf = bufs[ci % 2]
        row0 = base + ci * _CHUNK

        def row_body(r, carry, buf=buf, row0=row0):
            a_b, sq_b, bx = carry
            dps = [jnp.zeros((_L,), jnp.float32) for _ in range(4)]
            sps = [jnp.zeros((_L,), jnp.float32) for _ in range(4)]
            for j in range(_DCH):
                c = buf[r, pl.ds(j * _L, _L)]
                k = j % 4
                dps[k] = dps[k] + c * ech[j]
                sps[k] = sps[k] + c * c
            dot = (dps[0] + dps[1]) + (dps[2] + dps[3])
            sq = (sps[0] + sps[1]) + (sps[2] + sps[3])
            dotr = plsc.cumsum(dot)
            sqr = plsc.cumsum(sq)
            a = dotr * jnp.abs(dotr)
            better = a * sq_b > a_b * sqr
            ridx = jnp.broadcast_to(row0 + r, (_L,))
            a_b = jnp.where(better, a, a_b)
            sq_b = jnp.where(better, sqr, sq_b)
            bx = jnp.where(better, ridx, bx)
            return a_b, sq_b, bx

        a_b, sq_b, bx = lax.fori_loop(0, _CHUNK, row_body, (a_b, sq_b, bx))

    esq = jnp.zeros((_L,), jnp.float32)
    for j in range(_DCH):
        esq = esq + ech[j] * ech[j]
    en2_v = jnp.maximum(_lane15(plsc.cumsum(esq), fifteen), 1e-24)

    a15 = _lane15(a_b, fifteen)
    sq15 = _lane15(sq_b, fifteen)
    qf = jnp.clip(a15 / (jnp.maximum(sq15, 1e-24) * en2_v), 0.0, 1.0)
    st_f[...] = qf
    st_i[...] = _lane15(bx, fifteen)
    pltpu.sync_copy(st_f, sim_hbm.at[wid])
    pltpu.sync_copy(st_i, idx_hbm.at[wid])


_sc_call = pl.kernel(
    _sc_body,
    out_type=(jax.ShapeDtypeStruct((_NW, _L), jnp.float32),
              jax.ShapeDtypeStruct((_NW, _L), jnp.int32)),
    mesh=plsc.VectorSubcoreMesh(core_axis_name="c", subcore_axis_name="s"),
    scratch_types=[
        pltpu.VMEM((_D,), jnp.float32),
        pltpu.VMEM((_CHUNK, _D), jnp.float32),
        pltpu.VMEM((_CHUNK, _D), jnp.float32),
        pltpu.VMEM((_L,), jnp.float32),
        pltpu.VMEM((_L,), jnp.int32),
        pltpu.SemaphoreType.DMA,
        pltpu.SemaphoreType.DMA,
    ],
    compiler_params=pltpu.CompilerParams(needs_layout_passes=False),
)


def _merge_body(s_ref, i_ref, nov_ref, cls_ref):
    s = s_ref[...]
    i = i_ref[...]
    m = jnp.max(s)
    cand = jnp.where(s == m, i, _IMAX)
    max_sim = jnp.sqrt(m)
    nov_ref[0, 0] = jnp.minimum(jnp.sqrt(jnp.maximum(1.0 - max_sim, 0.0)), 1.0)
    cls_ref[0, 0] = jnp.where(m <= 0.0, np.int32(0), jnp.min(cand))


def kernel(embedding, centroids):
    sims, idxs = _sc_call(embedding, centroids)
    nov, cls = pl.pallas_call(
        _merge_body,
        out_shape=(jax.ShapeDtypeStruct((1, 1), jnp.float32),
                   jax.ShapeDtypeStruct((1, 1), jnp.int32)),
        out_specs=(pl.BlockSpec(memory_space=pltpu.SMEM),
                   pl.BlockSpec(memory_space=pltpu.SMEM)),
    )(sims, idxs)
    return nov.reshape(()), cls.reshape(())


# speedup vs baseline: 1.0369x; 1.0369x over previous
import jax
import jax.numpy as jnp
import numpy as np
from jax import lax
from jax.experimental import pallas as pl
from jax.experimental.pallas import tpu as pltpu
from jax.experimental.pallas import tpu_sc as plsc

_D = 256
_N = 8192
_NC = 2
_NS = 16
_NW = _NC * _NS
_L = 16
_ROWS = _N // _NW
_CHUNK = 64
_NCHUNK = _ROWS // _CHUNK
_DCH = _D // _L
_IMAX = np.int32(2**31 - 1)


def _lane15(x, fifteen):
    return x.at[fifteen].get(mode="promise_in_bounds")


def _sc_body(emb_hbm, cent_hbm, sim_hbm, idx_hbm,
             e_v, buf0, buf1, st_f, st_i, sem0, sem1):
    cid = lax.axis_index("c")
    sid = lax.axis_index("s")
    wid = cid * _NS + sid
    base = wid * _ROWS

    pltpu.sync_copy(emb_hbm, e_v)

    fifteen = jnp.broadcast_to(np.int32(_L - 1), (_L,))

    ech = [e_v[pl.ds(j * _L, _L)] for j in range(_DCH)]

    a_b = jnp.full((_L,), -jnp.inf, jnp.float32)
    sq_b = jnp.ones((_L,), jnp.float32)
    bx = jnp.zeros((_L,), jnp.int32)

    bufs = (buf0, buf1)
    sems = (sem0, sem1)
    copies = [None, None]
    copies[0] = pltpu.async_copy(cent_hbm.at[pl.ds(base, _CHUNK)], buf0, sem0)
    for ci in range(_NCHUNK):
        if ci + 1 < _NCHUNK:
            nxt = (ci + 1) % 2
            copies[nxt] = pltpu.async_copy(
                cent_hbm.at[pl.ds(base + (ci + 1) * _CHUNK, _CHUNK)],
                bufs[nxt], sems[nxt])
        copies[ci % 2].wait()
        buf = bufs[ci % 2]
        row0 = base + ci * _CHUNK

        def row_body(r, carry, buf=buf, row0=row0):
            a_b, sq_b, bx = carry
            dps = [jnp.zeros((_L,), jnp.float32) for _ in range(4)]
            sps = [jnp.zeros((_L,), jnp.float32) for _ in range(4)]
            for j in range(_DCH):
                c = buf[r, pl.ds(j * _L, _L)]
                k = j % 4
                dps[k] = dps[k] + c * ech[j]
                sps[k] = sps[k] + c * c
            dot = (dps[0] + dps[1]) + (dps[2] + dps[3])
            sq = (sps[0] + sps[1]) + (sps[2] + sps[3])
            dotr = plsc.cumsum(dot)
            sqr = plsc.cumsum(sq)
            a = dotr * jnp.abs(dotr)
            better = a * sq_b > a_b * sqr
            ridx = jnp.broadcast_to(row0 + r, (_L,))
            a_b = jnp.where(better, a, a_b)
            sq_b = jnp.where(better, sqr, sq_b)
            bx = jnp.where(better, ridx, bx)
            return a_b, sq_b, bx

        a_b, sq_b, bx = lax.fori_loop(0, _CHUNK, row_body, (a_b, sq_b, bx))

    esq = jnp.zeros((_L,), jnp.float32)
    for j in range(_DCH):
        esq = esq + ech[j] * ech[j]
    en2_v = jnp.maximum(_lane15(plsc.cumsum(esq), fifteen), 1e-24)

    a15 = _lane15(a_b, fifteen)
    sq15 = _lane15(sq_b, fifteen)
    qf = jnp.clip(a15 / (jnp.maximum(sq15, 1e-24) * en2_v), 0.0, 1.0)
    st_f[...] = qf
    st_i[...] = _lane15(bx, fifteen)
    pltpu.sync_copy(st_f, sim_hbm.at[wid])
    pltpu.sync_copy(st_i, idx_hbm.at[wid])


_sc_call = pl.kernel(
    _sc_body,
    out_type=(jax.ShapeDtypeStruct((_NW, _L), jnp.float32),
              jax.ShapeDtypeStruct((_NW, _L), jnp.int32)),
    mesh=plsc.VectorSubcoreMesh(core_axis_name="c", subcore_axis_name="s"),
    scratch_types=[
        pltpu.VMEM((_D,), jnp.float32),
        pltpu.VMEM((_CHUNK, _D), jnp.float32),
        pltpu.VMEM((_CHUNK, _D), jnp.float32),
        pltpu.VMEM((_L,), jnp.float32),
        pltpu.VMEM((_L,), jnp.int32),
        pltpu.SemaphoreType.DMA,
        pltpu.SemaphoreType.DMA,
    ],
    compiler_params=pltpu.CompilerParams(needs_layout_passes=False),
)


def _merge_body(s_ref, i_ref, nov_ref, cls_ref):
    s = s_ref[...]
    i = i_ref[...]
    m = jnp.max(s)
    cand = jnp.where(s == m, i, _IMAX)
    max_sim = jnp.sqrt(m)
    nov_ref[0, 0] = jnp.minimum(jnp.sqrt(jnp.maximum(1.0 - max_sim, 0.0)), 1.0)
    cls_ref[0, 0] = jnp.where(m <= 0.0, np.int32(0), jnp.min(cand))


def kernel(embedding, centroids):
    sims, idxs = _sc_call(embedding, centroids)
    nov, cls = pl.pallas_call(
        _merge_body,
        out_shape=(jax.ShapeDtypeStruct((1, 1), jnp.float32),
                   jax.ShapeDtypeStruct((1, 1), jnp.int32)),
        out_specs=(pl.BlockSpec(memory_space=pltpu.SMEM),
                   pl.BlockSpec(memory_space=pltpu.SMEM)),
    )(sims, idxs)
    return nov.reshape(()), cls.reshape(())
